# baseline (speedup 1.0000x reference)
import functools

import jax
import jax.numpy as jnp
from jax import lax
from jax.experimental import pallas as pl
from jax.experimental.pallas import tpu as pltpu
from jax.experimental.pallas import tpu_sc as plsc

N_NODES = 100000
D_FEAT = 128
NUM_GRAPHS = 64
NUM_WORKERS = 32
NLANES = 16
NVEC = D_FEAT // NLANES

N_SC = 42400
TC_BLK = 800
TC_RED = 16
N_TC = N_NODES - N_SC
N_BMAX = N_TC // TC_RED

RPW_A = 1344
STRIDE_A = 1328
BASE_MAX_A = N_SC - RPW_A
CHUNK = 224
NUM_CHUNKS = RPW_A // CHUNK

RPW_C = 128
STRIDE_C = 112
BASE_MAX_C = N_BMAX - RPW_C


def _sc_stage_a(feats, ids):
    mesh = plsc.VectorSubcoreMesh(core_axis_name="c", subcore_axis_name="s")

    @functools.partial(
        pl.kernel,
        out_type=jax.ShapeDtypeStruct((NUM_WORKERS, NUM_GRAPHS, D_FEAT),
                                      jnp.float32),
        mesh=mesh,
        scratch_types=[
            pltpu.VMEM((CHUNK, D_FEAT), jnp.float32),
            pltpu.VMEM((CHUNK, D_FEAT), jnp.float32),
            pltpu.VMEM((CHUNK,), jnp.int32),
            pltpu.VMEM((CHUNK,), jnp.int32),
            pltpu.VMEM((NUM_GRAPHS, D_FEAT), jnp.float32),
            pltpu.SemaphoreType.DMA,
            pltpu.SemaphoreType.DMA,
            pltpu.SemaphoreType.DMA,
            pltpu.SemaphoreType.DMA,
        ],
    )
    def k(feat_hbm, ids_hbm, out_hbm, fb0, fb1, ib0, ib1, tab,
          fs0, fs1, is0, is1):
        wid = lax.axis_index("s") * 2 + lax.axis_index("c")
        base = jnp.minimum(wid * STRIDE_A, BASE_MAX_A)
        fbuf = (fb0, fb1)
        ibuf = (ib0, ib1)
        fsem = (fs0, fs1)
        isem = (is0, is1)

        def start(c, b):
            off = base + c * CHUNK
            pltpu.async_copy(feat_hbm.at[pl.ds(off, CHUNK), :], fbuf[b],
                             fsem[b])
            pltpu.async_copy(ids_hbm.at[pl.ds(off, CHUNK)], ibuf[b],
                             isem[b])

        def wait(b):
            pltpu.make_async_copy(feat_hbm.at[pl.ds(0, CHUNK), :], fbuf[b],
                                  fsem[b]).wait()
            pltpu.make_async_copy(ids_hbm.at[pl.ds(0, CHUNK)], ibuf[b],
                                  isem[b]).wait()

        start(0, 0)
        start(1, 1)

        neg = jnp.full((NLANES,), -jnp.inf, dtype=jnp.float32)

        def init_body(g, _):
            for j in range(NVEC):
                tab[g, pl.ds(j * NLANES, NLANES)] = neg
            return 0

        lax.fori_loop(0, NUM_GRAPHS, init_body, 0)

        def tree_max(bref, r0, sl):
            m = [bref[r0 + i, sl] for i in range(NLANES)]
            while len(m) > 1:
                m = ([jnp.maximum(m[2 * t], m[2 * t + 1])
                      for t in range(len(m) // 2)]
                     + m[len(m) // 2 * 2:])
            return m[0]

        def compute(b):
            def fast16(r0, g0):
                for j in range(NVEC):
                    sl = pl.ds(j * NLANES, NLANES)
                    tab[g0, sl] = jnp.maximum(tab[g0, sl],
                                              tree_max(fbuf[b], r0, sl))

            def slow16(r0, idvec):
                for i in range(NLANES):
                    g = idvec[i]
                    for j in range(NVEC):
                        sl = pl.ds(j * NLANES, NLANES)
                        tab[g, sl] = jnp.maximum(tab[g, sl],
                                                 fbuf[b][r0 + i, sl])

            def half(r0, idvec):
                lax.cond(idvec[0] == idvec[NLANES - 1],
                         lambda _: (fast16(r0, idvec[0]), 0)[1],
                         lambda _: (slow16(r0, idvec), 0)[1], 0)

            def grp_body(q, _):
                r0 = q * 2 * NLANES
                idv0 = ibuf[b][pl.ds(r0, NLANES)]
                idv1 = ibuf[b][pl.ds(r0 + NLANES, NLANES)]
                g0 = idv0[0]

                def fast32(_):
                    for j in range(NVEC):
                        sl = pl.ds(j * NLANES, NLANES)
                        v = jnp.maximum(tree_max(fbuf[b], r0, sl),
                                        tree_max(fbuf[b], r0 + NLANES, sl))
                        tab[g0, sl] = jnp.maximum(tab[g0, sl], v)
                    return 0

                def slow32(_):
                    half(r0, idv0)
                    half(r0 + NLANES, idv1)
                    return 0

                lax.cond(g0 == idv1[NLANES - 1], fast32, slow32, 0)
                return 0

            lax.fori_loop(0, CHUNK // (2 * NLANES), grp_body, 0)

        def body2(cc, _):
            for b in range(2):
                c = 2 * cc + b
                wait(b)
                compute(b)

                @pl.when(c + 2 < NUM_CHUNKS)
                def _():
                    start(c + 2, b)
            return 0

        lax.fori_loop(0, NUM_CHUNKS // 2, body2, 0)
        pltpu.sync_copy(tab, out_hbm.at[wid])

    return k(feats, ids)


def _tc_block_reduce(feats3):
    def body(f_ref, o_ref):
        x = f_ref[0].reshape(TC_BLK // TC_RED, TC_RED, D_FEAT)
        o_ref[0] = jnp.max(x, axis=1)

    nblk = N_TC // TC_BLK
    off = N_SC // TC_BLK
    return pl.pallas_call(
        body,
        grid=(nblk,),
        in_specs=[pl.BlockSpec((1, TC_BLK, D_FEAT),
                               lambda i: (off + i, 0, 0))],
        out_specs=pl.BlockSpec((1, TC_BLK // TC_RED, D_FEAT),
                               lambda i: (i, 0, 0)),
        out_shape=jax.ShapeDtypeStruct((nblk, TC_BLK // TC_RED, D_FEAT),
                                       jnp.float32),
    )(feats3)


def _sc_stage_c(bmax, ids, feats):
    mesh = plsc.VectorSubcoreMesh(core_axis_name="c", subcore_axis_name="s")

    @functools.partial(
        pl.kernel,
        out_type=jax.ShapeDtypeStruct((NUM_WORKERS, NUM_GRAPHS, D_FEAT),
                                      jnp.float32),
        mesh=mesh,
        scratch_types=[
            pltpu.VMEM((RPW_C, D_FEAT), jnp.float32),
            pltpu.VMEM((RPW_C * TC_RED,), jnp.int32),
            pltpu.VMEM((TC_RED, D_FEAT), jnp.float32),
            pltpu.VMEM((NUM_GRAPHS, D_FEAT), jnp.float32),
            pltpu.SemaphoreType.DMA,
            pltpu.SemaphoreType.DMA,
        ],
    )
    def k(bmax_hbm, ids_hbm, feat_hbm, out_hbm, bbuf, idsl, raw, tab,
          bsem, isem):
        wid = lax.axis_index("s") * 2 + lax.axis_index("c")
        base = jnp.minimum(wid * STRIDE_C, BASE_MAX_C)

        pltpu.async_copy(bmax_hbm.at[pl.ds(base, RPW_C), :], bbuf, bsem)
        pltpu.async_copy(ids_hbm.at[pl.ds(N_SC + base * TC_RED,
                                          RPW_C * TC_RED)], idsl, isem)

        neg = jnp.full((NLANES,), -jnp.inf, dtype=jnp.float32)

        def init_body(g, _):
            for j in range(NVEC):
                tab[g, pl.ds(j * NLANES, NLANES)] = neg
            return 0

        lax.fori_loop(0, NUM_GRAPHS, init_body, 0)
        pltpu.make_async_copy(bmax_hbm.at[pl.ds(0, RPW_C), :], bbuf,
                              bsem).wait()
        pltpu.make_async_copy(ids_hbm.at[pl.ds(0, RPW_C * TC_RED)], idsl,
                              isem).wait()

        def tree_max(bref, r0, sl):
            m = [bref[r0 + i, sl] for i in range(NLANES)]
            while len(m) > 1:
                m = ([jnp.maximum(m[2 * t], m[2 * t + 1])
                      for t in range(len(m) // 2)]
                     + m[len(m) // 2 * 2:])
            return m[0]

        def grp_body(q, _):
            r0 = q * NLANES
            va = idsl[pl.ds(r0 * TC_RED, NLANES)]
            vb = idsl[pl.ds(r0 * TC_RED + 255 - (NLANES - 1), NLANES)]
            g0 = va[0]

            def fastq(_):
                for j in range(NVEC):
                    sl = pl.ds(j * NLANES, NLANES)
                    tab[g0, sl] = jnp.maximum(tab[g0, sl],
                                              tree_max(bbuf, r0, sl))
                return 0

            def slowq(_):
                def row_body(kk, _):
                    idv = idsl[pl.ds((r0 + kk) * TC_RED, NLANES)]
                    u0 = idv[0]

                    def row_fast(_):
                        for j in range(NVEC):
                            sl = pl.ds(j * NLANES, NLANES)
                            tab[u0, sl] = jnp.maximum(tab[u0, sl],
                                                      bbuf[r0 + kk, sl])
                        return 0

                    def row_slow(_):
                        off = N_SC + (base + r0 + kk) * TC_RED
                        pltpu.sync_copy(feat_hbm.at[pl.ds(off, TC_RED), :],
                                        raw)
                        for i in range(TC_RED):
                            g = idv[i]
                            for j in range(NVEC):
                                sl = pl.ds(j * NLANES, NLANES)
                                tab[g, sl] = jnp.maximum(tab[g, sl],
                                                         raw[i, sl])
                        return 0

                    lax.cond(u0 == idv[NLANES - 1], row_fast, row_slow, 0)
                    return 0

                lax.fori_loop(0, NLANES, row_body, 0)
                return 0

            lax.cond(g0 == vb[NLANES - 1], fastq, slowq, 0)
            return 0

        lax.fori_loop(0, RPW_C // NLANES, grp_body, 0)
        pltpu.sync_copy(tab, out_hbm.at[wid])

    return k(bmax, ids, feats)


def _combine_body(a_ref, b_ref, o_ref):
    o_ref[...] = jnp.maximum(jnp.max(a_ref[...], axis=0),
                             jnp.max(b_ref[...], axis=0))


def kernel(features_0, graph_ids):
    feats = features_0.reshape(N_NODES, D_FEAT)
    feats3 = features_0.reshape(N_NODES // TC_BLK, TC_BLK, D_FEAT)
    ids = graph_ids.astype(jnp.int32)
    p_a = _sc_stage_a(feats, ids)
    bmax = _tc_block_reduce(feats3).reshape(N_BMAX, D_FEAT)
    p_c = _sc_stage_c(bmax, ids, feats)
    out = pl.pallas_call(
        _combine_body,
        out_shape=jax.ShapeDtypeStruct((NUM_GRAPHS, D_FEAT), jnp.float32),
    )(p_a, p_c)
    return out


# speedup vs baseline: 1.7682x; 1.7682x over previous
import functools

import jax
import jax.numpy as jnp
from jax import lax
from jax.experimental import pallas as pl
from jax.experimental.pallas import tpu as pltpu
from jax.experimental.pallas import tpu_sc as plsc

N_NODES = 100000
D_FEAT = 128
NUM_GRAPHS = 64
NUM_WORKERS = 32
ROWS_PER_WORKER = 3136
CHUNK = 224
NUM_CHUNKS = ROWS_PER_WORKER // CHUNK
NLANES = 16
NVEC = D_FEAT // NLANES


def _sc_partials(feats, ids):
    mesh = plsc.VectorSubcoreMesh(core_axis_name="c", subcore_axis_name="s")

    @functools.partial(
        pl.kernel,
        out_type=jax.ShapeDtypeStruct((NUM_WORKERS, NUM_GRAPHS, D_FEAT),
                                      jnp.float32),
        mesh=mesh,
        scratch_types=[
            pltpu.VMEM((CHUNK, D_FEAT), jnp.float32),
            pltpu.VMEM((CHUNK, D_FEAT), jnp.float32),
            pltpu.VMEM((CHUNK,), jnp.int32),
            pltpu.VMEM((CHUNK,), jnp.int32),
            pltpu.VMEM((NUM_GRAPHS, D_FEAT), jnp.float32),
            pltpu.SemaphoreType.DMA,
            pltpu.SemaphoreType.DMA,
            pltpu.SemaphoreType.DMA,
            pltpu.SemaphoreType.DMA,
        ],
    )
    def k(feat_hbm, ids_hbm, out_hbm, fb0, fb1, ib0, ib1, tab,
          fs0, fs1, is0, is1):
        wid = lax.axis_index("s") * 2 + lax.axis_index("c")
        base = jnp.minimum(wid * ROWS_PER_WORKER, N_NODES - ROWS_PER_WORKER)
        fbuf = (fb0, fb1)
        ibuf = (ib0, ib1)
        fsem = (fs0, fs1)
        isem = (is0, is1)

        def start(c, b):
            off = base + c * CHUNK
            pltpu.async_copy(feat_hbm.at[pl.ds(off, CHUNK), :], fbuf[b],
                             fsem[b])
            pltpu.async_copy(ids_hbm.at[pl.ds(off, CHUNK)], ibuf[b],
                             isem[b])

        def wait(b):
            pltpu.make_async_copy(feat_hbm.at[pl.ds(0, CHUNK), :], fbuf[b],
                                  fsem[b]).wait()
            pltpu.make_async_copy(ids_hbm.at[pl.ds(0, CHUNK)], ibuf[b],
                                  isem[b]).wait()

        start(0, 0)
        start(1, 1)

        neg = jnp.full((NLANES,), -jnp.inf, dtype=jnp.float32)

        def init_body(g, _):
            for j in range(NVEC):
                tab[g, pl.ds(j * NLANES, NLANES)] = neg
            return 0

        lax.fori_loop(0, NUM_GRAPHS, init_body, 0)

        def compute(b):
            def grp_body(q, _):
                r0 = q * NLANES
                idvec = ibuf[b][pl.ds(r0, NLANES)]
                g0 = idvec[0]

                def fast(_):
                    for j in range(NVEC):
                        sl = pl.ds(j * NLANES, NLANES)
                        m = [fbuf[b][r0 + i, sl] for i in range(NLANES)]
                        while len(m) > 1:
                            m = ([jnp.maximum(m[2 * t], m[2 * t + 1])
                                  for t in range(len(m) // 2)]
                                 + m[len(m) // 2 * 2:])
                        tab[g0, sl] = jnp.maximum(tab[g0, sl], m[0])
                    return 0

                def slow(_):
                    for i in range(NLANES):
                        g = idvec[i]
                        for j in range(NVEC):
                            sl = pl.ds(j * NLANES, NLANES)
                            tab[g, sl] = jnp.maximum(tab[g, sl],
                                                     fbuf[b][r0 + i, sl])
                    return 0

                lax.cond(g0 == idvec[NLANES - 1], fast, slow, 0)
                return 0

            lax.fori_loop(0, CHUNK // NLANES, grp_body, 0)

        def body2(cc, _):
            for b in range(2):
                c = 2 * cc + b
                wait(b)
                compute(b)

                @pl.when(c + 2 < NUM_CHUNKS)
                def _():
                    start(c + 2, b)
            return 0

        lax.fori_loop(0, NUM_CHUNKS // 2, body2, 0)
        if NUM_CHUNKS % 2:
            wait((NUM_CHUNKS - 1) % 2)
            compute((NUM_CHUNKS - 1) % 2)
        pltpu.sync_copy(tab, out_hbm.at[wid])

    return k(feats, ids)


def _reduce_body(p_ref, o_ref):
    o_ref[...] = jnp.max(p_ref[...], axis=0)


def kernel(features_0, graph_ids):
    feats = features_0.reshape(N_NODES, D_FEAT)
    ids = graph_ids.astype(jnp.int32)
    partials = _sc_partials(feats, ids)
    out = pl.pallas_call(
        _reduce_body,
        out_shape=jax.ShapeDtypeStruct((NUM_GRAPHS, D_FEAT), jnp.float32),
    )(partials)
    return out
